# baseline (speedup 1.0000x reference)
import functools
import math

import jax
import jax.numpy as jnp
from jax import lax
from jax.experimental import pallas as pl
from jax.experimental.pallas import tpu as pltpu
from jax.experimental.pallas import tpu_sc as plsc

D_MODEL = 64
SCALE = math.sqrt(D_MODEL)

NUM_CORES = 2
NUM_SUBCORES = 16
NUM_WORKERS = NUM_CORES * NUM_SUBCORES

BBLK = 128


def _make_embed(batch: int, seq: int):
    assert batch == NUM_WORKERS * BBLK
    assert seq % 2 == 0
    n_tiles_c = D_MODEL // 8
    n_tiles_b = batch // BBLK

    mesh = plsc.VectorSubcoreMesh(
        core_axis_name="c", subcore_axis_name="s"
    )

    @functools.partial(
        pl.kernel,
        out_type=jax.ShapeDtypeStruct(
            (seq, n_tiles_c, n_tiles_b, 8, BBLK), jnp.float32
        ),
        mesh=mesh,
        scratch_types=[
            pltpu.VMEM((seq, BBLK), jnp.int32),
            pltpu.VMEM((BBLK, D_MODEL), jnp.float32),
            pltpu.VMEM((BBLK, D_MODEL), jnp.float32),
            pltpu.VMEM((BBLK, D_MODEL), jnp.float32),
            pltpu.VMEM((BBLK, D_MODEL), jnp.float32),
            pltpu.VMEM((D_MODEL, BBLK + 1), jnp.float32),
            pltpu.VMEM((D_MODEL, BBLK + 1), jnp.float32),
            pltpu.VMEM((D_MODEL, BBLK + 1), jnp.float32),
            pltpu.VMEM((D_MODEL, BBLK + 1), jnp.float32),
            pltpu.SemaphoreType.DMA,
            pltpu.SemaphoreType.DMA,
            pltpu.SemaphoreType.DMA,
            pltpu.SemaphoreType.DMA,
            pltpu.SemaphoreType.DMA,
            pltpu.SemaphoreType.DMA,
            pltpu.SemaphoreType.DMA,
            pltpu.SemaphoreType.DMA,
        ],
        compiler_params=pltpu.CompilerParams(
            use_tc_tiling_on_sc=False, needs_layout_passes=False
        ),
    )
    def embed(table_hbm, xt_hbm, out_hbm, idx_all,
              rows0, rows1, rows2, rows3, tr0, tr1, tr2, tr3,
              gsem0, gsem1, gsem2, gsem3, ssem0, ssem1, ssem2, ssem3):
        wid = lax.axis_index("s") * NUM_CORES + lax.axis_index("c")
        b0 = wid * BBLK
        rows = (rows0, rows1, rows2, rows3)
        trs = (tr0, tr1, tr2, tr3)
        gsems = (gsem0, gsem1, gsem2, gsem3)
        ssems = (ssem0, ssem1, ssem2, ssem3)

        pltpu.sync_copy(xt_hbm.at[:, pl.ds(b0, BBLK)], idx_all)

        def start_gather(s, b):
            pltpu.async_copy(
                table_hbm.at[idx_all.at[s]], rows[b], gsems[b]
            )

        def wait_gather(b):
            pltpu.make_async_copy(
                table_hbm.at[idx_all.at[0]], rows[b], gsems[b]
            ).wait()

        def start_scatter(s, b):
            for r in range(n_tiles_c):
                pltpu.async_copy(
                    trs[b].at[pl.ds(r * 8, 8), pl.ds(0, BBLK)],
                    out_hbm.at[s, r, wid],
                    ssems[b],
                )

        def wait_scatter(b):
            for r in range(n_tiles_c):
                pltpu.make_async_copy(
                    trs[b].at[pl.ds(r * 8, 8), pl.ds(0, BBLK)],
                    out_hbm.at[0, r, wid],
                    ssems[b],
                ).wait()

        cvecs = [g * 16 + lax.iota(jnp.int32, 16) for g in range(D_MODEL // 16)]

        def transpose_scale(b):
            src = rows[b]
            dst = trs[b]

            @plsc.parallel_loop(0, BBLK, 1, unroll=8)
            def _(r):
                r16 = jnp.full((16,), r, dtype=jnp.int32)
                for g in range(D_MODEL // 16):
                    v = src[r, pl.ds(g * 16, 16)]
                    plsc.store_scatter(dst, [cvecs[g], r16], v * SCALE)

        start_gather(0, 0)
        start_gather(1, 1)

        def quad_body(p, carry):
            for b in range(4):
                s = 4 * p + b
                nb = (b + 2) % 4

                @pl.when(s < seq - 2)
                def _():
                    start_gather(s + 2, nb)

                wait_gather(b)

                @pl.when(s >= 4)
                def _():
                    wait_scatter(b)

                transpose_scale(b)
                start_scatter(s, b)
            return carry

        lax.fori_loop(0, seq // 4, quad_body, 0)
        for b in range(4):
            wait_scatter(b)

    return embed


def kernel(x, table):
    batch, seq = x.shape
    out5 = _make_embed(batch, seq)(table, x.T.astype(jnp.int32))
    return out5.transpose(2, 4, 0, 1, 3).reshape(batch, seq, D_MODEL)


# speedup vs baseline: 1.0094x; 1.0094x over previous
import functools
import math

import jax
import jax.numpy as jnp
from jax import lax
from jax.experimental import pallas as pl
from jax.experimental.pallas import tpu as pltpu
from jax.experimental.pallas import tpu_sc as plsc

D_MODEL = 64
SCALE = math.sqrt(D_MODEL)

NUM_CORES = 2
NUM_SUBCORES = 16
NUM_WORKERS = NUM_CORES * NUM_SUBCORES

BBLK = 128


def _make_embed(batch: int, seq: int):
    assert batch == NUM_WORKERS * BBLK
    assert seq % 2 == 0
    n_tiles_c = D_MODEL // 8
    n_tiles_b = batch // BBLK

    mesh = plsc.VectorSubcoreMesh(
        core_axis_name="c", subcore_axis_name="s"
    )

    @functools.partial(
        pl.kernel,
        out_type=jax.ShapeDtypeStruct(
            (seq, n_tiles_c, n_tiles_b, 8, BBLK), jnp.float32
        ),
        mesh=mesh,
        scratch_types=[
            pltpu.VMEM((seq, BBLK), jnp.int32),
            pltpu.VMEM((BBLK, D_MODEL), jnp.float32),
            pltpu.VMEM((BBLK, D_MODEL), jnp.float32),
            pltpu.VMEM((BBLK, D_MODEL), jnp.float32),
            pltpu.VMEM((BBLK, D_MODEL), jnp.float32),
            pltpu.VMEM((D_MODEL, BBLK + 1), jnp.float32),
            pltpu.VMEM((D_MODEL, BBLK + 1), jnp.float32),
            pltpu.VMEM((D_MODEL, BBLK + 1), jnp.float32),
            pltpu.VMEM((D_MODEL, BBLK + 1), jnp.float32),
            pltpu.SemaphoreType.DMA,
            pltpu.SemaphoreType.DMA,
            pltpu.SemaphoreType.DMA,
            pltpu.SemaphoreType.DMA,
            pltpu.SemaphoreType.DMA,
            pltpu.SemaphoreType.DMA,
            pltpu.SemaphoreType.DMA,
            pltpu.SemaphoreType.DMA,
        ],
        compiler_params=pltpu.CompilerParams(
            use_tc_tiling_on_sc=False, needs_layout_passes=False
        ),
    )
    def embed(table_hbm, xt_hbm, out_hbm, idx_all,
              rows0, rows1, rows2, rows3, tr0, tr1, tr2, tr3,
              gsem0, gsem1, gsem2, gsem3, ssem0, ssem1, ssem2, ssem3):
        wid = lax.axis_index("s") * NUM_CORES + lax.axis_index("c")
        b0 = wid * BBLK
        rows = (rows0, rows1, rows2, rows3)
        trs = (tr0, tr1, tr2, tr3)
        gsems = (gsem0, gsem1, gsem2, gsem3)
        ssems = (ssem0, ssem1, ssem2, ssem3)

        pltpu.sync_copy(xt_hbm.at[:, pl.ds(b0, BBLK)], idx_all)

        def start_gather(s, b):
            pltpu.async_copy(
                table_hbm.at[idx_all.at[s]], rows[b], gsems[b]
            )

        def wait_gather(b):
            pltpu.make_async_copy(
                table_hbm.at[idx_all.at[0]], rows[b], gsems[b]
            ).wait()

        def start_scatter(s, b):
            for r in range(n_tiles_c):
                pltpu.async_copy(
                    trs[b].at[pl.ds(r * 8, 8), pl.ds(0, BBLK)],
                    out_hbm.at[s, r, wid],
                    ssems[b],
                )

        def wait_scatter(b):
            for r in range(n_tiles_c):
                pltpu.make_async_copy(
                    trs[b].at[pl.ds(r * 8, 8), pl.ds(0, BBLK)],
                    out_hbm.at[0, r, wid],
                    ssems[b],
                ).wait()

        cvecs = [g * 16 + lax.iota(jnp.int32, 16) for g in range(D_MODEL // 16)]

        def transpose_scale(b):
            src = rows[b]
            dst = trs[b]

            @plsc.parallel_loop(0, BBLK, 1, unroll=4)
            def _(r):
                r16 = jnp.full((16,), r, dtype=jnp.int32)
                for g in range(D_MODEL // 16):
                    v = src[r, pl.ds(g * 16, 16)]
                    plsc.store_scatter(dst, [cvecs[g], r16], v * SCALE)

        start_gather(0, 0)
        start_gather(1, 1)
        start_gather(2, 2)

        def quad_body(p, carry):
            for b in range(4):
                s = 4 * p + b
                nb = (b + 3) % 4

                @pl.when(s < seq - 3)
                def _():
                    start_gather(s + 3, nb)

                wait_gather(b)

                @pl.when(s >= 4)
                def _():
                    wait_scatter(b)

                transpose_scale(b)
                start_scatter(s, b)
            return carry

        lax.fori_loop(0, seq // 4, quad_body, 0)
        for b in range(4):
            wait_scatter(b)

    return embed


def kernel(x, table):
    batch, seq = x.shape
    out5 = _make_embed(batch, seq)(table, x.T.astype(jnp.int32))
    return out5.transpose(2, 4, 0, 1, 3).reshape(batch, seq, D_MODEL)
